# baseline (speedup 1.0000x reference)
import jax
import jax.numpy as jnp
from jax import lax
from jax.experimental import pallas as pl
from jax.experimental.pallas import tpu as pltpu
from jax.experimental.pallas import tpu_sc as plsc

NC = 2
NS = 16
CHUNK = 128
K = 2
NB = 2 * K

MLP_BLK = 2000


def _mlp1_body(x_ref, w1_ref, b1_ref, w2_ref, b2_ref, o_ref):
    h = jnp.dot(x_ref[...], w1_ref[...], preferred_element_type=jnp.float32) + b1_ref[...]
    h = jnp.maximum(h, 0.0)
    o = jnp.dot(h, w2_ref[...], preferred_element_type=jnp.float32) + b2_ref[...]
    d2 = o.shape[1] // 2
    o_ref[0] = o[:, :d2]
    o_ref[1] = o[:, d2:]


def _mlp1(x, w1, b1, w2, b2, a_pad):
    n, d_in = x.shape
    hid = w1.shape[1]
    d_out = w2.shape[1]
    grid = n // MLP_BLK
    return pl.pallas_call(
        _mlp1_body,
        grid=(grid,),
        in_specs=[
            pl.BlockSpec((MLP_BLK, d_in), lambda i: (i, 0)),
            pl.BlockSpec((d_in, hid), lambda i: (0, 0)),
            pl.BlockSpec((1, hid), lambda i: (0, 0)),
            pl.BlockSpec((hid, d_out), lambda i: (0, 0)),
            pl.BlockSpec((1, d_out), lambda i: (0, 0)),
        ],
        out_specs=pl.BlockSpec((2, MLP_BLK, d_out // 2), lambda i: (0, i, 0)),
        out_shape=jax.ShapeDtypeStruct((2, a_pad, d_out // 2), jnp.float32),
    )(x, w1, b1, w2, b2)


def _mlp2_body(p_ref, cf_ref, w1a_ref, w1b_ref, b1_ref, w2_ref, b2_ref, o_ref):
    cembs = jnp.concatenate(
        [jnp.maximum(p_ref[0], 0.0), jnp.maximum(p_ref[1], 0.0)], axis=1)
    h = (jnp.dot(cembs, w1a_ref[...], preferred_element_type=jnp.float32)
         + jnp.dot(cf_ref[...], w1b_ref[...], preferred_element_type=jnp.float32)
         + b1_ref[...])
    h = jnp.maximum(h, 0.0)
    o = jnp.dot(h, w2_ref[...], preferred_element_type=jnp.float32) + b2_ref[...]
    d2 = o.shape[1] // 2
    o_ref[0] = o[:, :d2]
    o_ref[1] = o[:, d2:]


def _mlp2(partials, cf, w1a, w1b, b1, w2, b2, a_pad):
    d2 = partials.shape[2]
    n = cf.shape[0]
    dc = cf.shape[1]
    hid = w1a.shape[1]
    d_out = w2.shape[1]
    grid = n // MLP_BLK
    return pl.pallas_call(
        _mlp2_body,
        grid=(grid,),
        in_specs=[
            pl.BlockSpec((2, MLP_BLK, d2), lambda i: (0, i, 0)),
            pl.BlockSpec((MLP_BLK, dc), lambda i: (i, 0)),
            pl.BlockSpec((2 * d2, hid), lambda i: (0, 0)),
            pl.BlockSpec((dc, hid), lambda i: (0, 0)),
            pl.BlockSpec((1, hid), lambda i: (0, 0)),
            pl.BlockSpec((hid, d_out), lambda i: (0, 0)),
            pl.BlockSpec((1, d_out), lambda i: (0, 0)),
        ],
        out_specs=pl.BlockSpec((2, MLP_BLK, d_out // 2), lambda i: (0, i, 0)),
        out_shape=jax.ShapeDtypeStruct((2, a_pad, d_out // 2), jnp.float32),
    )(partials, cf, w1a, w1b, b1, w2, b2)


def _relu_halves_body(p_ref, o_ref):
    d2 = p_ref.shape[2]
    o_ref[:, :d2] = jnp.maximum(p_ref[0], 0.0)
    o_ref[:, d2:] = jnp.maximum(p_ref[1], 0.0)


def _relu_halves(partials, n):
    d2 = partials.shape[2]
    grid = n // MLP_BLK
    return pl.pallas_call(
        _relu_halves_body,
        grid=(grid,),
        in_specs=[pl.BlockSpec((2, MLP_BLK, d2), lambda i: (0, i, 0))],
        out_specs=pl.BlockSpec((MLP_BLK, 2 * d2), lambda i: (i, 0)),
        out_shape=jax.ShapeDtypeStruct((n, 2 * d2), jnp.float32),
    )(partials)


def _edge_pass(table, g_idx, s_idx, zeros, a_pad):
    d2 = table.shape[2]
    chunks = g_idx.shape[1]
    z_rows = a_pad // NS
    nsub = 4
    qch = chunks // nsub

    def body(table_hbm, g_hbm, s_hbm, z_hbm, out_hbm,
             table_sp, accum, g_v, s_v, rows_v, gsem, ssem):
        c = lax.axis_index("c")
        s = lax.axis_index("s")
        pltpu.sync_copy(z_hbm, accum.at[pl.ds(s * z_rows, z_rows)])
        pltpu.sync_copy(table_hbm.at[c, pl.ds(s * z_rows, z_rows)],
                        table_sp.at[pl.ds(s * z_rows, z_rows)])
        plsc.subcore_barrier()

        def fire_gather(j, b):
            pltpu.async_copy(table_sp.at[g_v.at[j]], rows_v.at[b], gsem)

        def wait_gather(b):
            pltpu.make_async_copy(table_sp.at[g_v.at[0]],
                                  rows_v.at[b], gsem).wait()

        def fire_scatter(j, b):
            pltpu.async_copy(rows_v.at[b], accum.at[s_v.at[j]], ssem,
                             add=True)

        def drain_scatter(b):
            pltpu.make_async_copy(rows_v.at[b], accum.at[s_v.at[0]],
                                  ssem).wait()

        def ring_step(t4, carry):
            for r in range(NB):
                t = t4 * NB + r
                wait_gather(r)
                fire_scatter(t, r)

                @pl.when(t >= 2)
                def _():
                    drain_scatter((r + 2) % NB)

                @pl.when(t + 2 < qch)
                def _():
                    fire_gather(t + 2, (r + 2) % NB)
            return carry

        for q in range(nsub):
            base = q * qch
            pltpu.sync_copy(g_hbm.at[s, pl.ds(base, qch)], g_v)
            pltpu.sync_copy(s_hbm.at[s, pl.ds(base, qch)], s_v)
            fire_gather(0, 0)
            fire_gather(1, 1)
            lax.fori_loop(0, qch // NB, ring_step, 0)
            drain_scatter(0)
            drain_scatter(1)

        plsc.subcore_barrier()
        pltpu.sync_copy(accum.at[pl.ds(s * z_rows, z_rows)],
                        out_hbm.at[c, pl.ds(s * z_rows, z_rows)])

    return pl.kernel(
        body,
        out_type=jax.ShapeDtypeStruct((NC, a_pad, d2), jnp.float32),
        mesh=plsc.VectorSubcoreMesh(core_axis_name="c", subcore_axis_name="s"),
        compiler_params=pltpu.CompilerParams(use_tc_tiling_on_sc=False),
        scratch_types=[
            pltpu.VMEM_SHARED((a_pad, d2), jnp.float32),
            pltpu.VMEM_SHARED((a_pad, d2), jnp.float32),
            pltpu.VMEM((chunks // nsub, CHUNK), jnp.int32),
            pltpu.VMEM((chunks // nsub, CHUNK), jnp.int32),
            pltpu.VMEM((NB, CHUNK, d2), jnp.float32),
            pltpu.SemaphoreType.DMA,
            pltpu.SemaphoreType.DMA,
        ],
    )(table, g_idx, s_idx, zeros)


def kernel(literal_feat, clause_feat, edge_index,
           W1_l2c, b1_l2c, W2_l2c, b2_l2c,
           W1_c2l, b1_c2l, W2_c2l, b2_c2l):
    n_lit, d_in = literal_feat.shape
    n_cl = clause_feat.shape[0]
    e = edge_index.shape[1]
    d = W2_l2c.shape[1]
    d2 = d // 2

    grain = NS * CHUNK * 2 * K * 4
    e_pad = -(-e // grain) * grain
    chunks = e_pad // (NS * CHUNK)
    a_pad = -(-(max(n_lit, n_cl) + 1) // (NS * 8)) * (NS * 8)
    dummy = max(n_lit, n_cl)

    src = edge_index[0].astype(jnp.int32)
    dst = edge_index[1].astype(jnp.int32)
    pad = e_pad - e
    dpad = jnp.full((pad,), dummy, jnp.int32)
    srcp = jnp.concatenate([src, dpad]).reshape(NS, chunks, CHUNK)
    dstp = jnp.concatenate([dst, dpad]).reshape(NS, chunks, CHUNK)
    zeros = jnp.zeros((a_pad // NS, d2), jnp.float32)

    wh_l2c = _mlp1(literal_feat, W1_l2c, b1_l2c.reshape(1, -1),
                   W2_l2c, b2_l2c.reshape(1, -1), a_pad)
    p_cl = _edge_pass(wh_l2c, srcp, dstp, zeros, a_pad)
    wh_c2l = _mlp2(p_cl, clause_feat,
                   W1_c2l[:d], W1_c2l[d:],
                   b1_c2l.reshape(1, -1), W2_c2l, b2_c2l.reshape(1, -1), a_pad)
    p_lit = _edge_pass(wh_c2l, dstp, srcp, zeros, a_pad)
    return _relu_halves(p_lit, n_lit)


# speedup vs baseline: 1.0189x; 1.0189x over previous
import jax
import jax.numpy as jnp
from jax import lax
from jax.experimental import pallas as pl
from jax.experimental.pallas import tpu as pltpu
from jax.experimental.pallas import tpu_sc as plsc

NC = 2
NS = 16
CHUNK = 128
K = 2
NB = 2 * K

MLP_BLK = 2000


def _mlp1_body(x_ref, w1_ref, b1_ref, w2_ref, b2_ref, o_ref):
    h = jnp.dot(x_ref[...], w1_ref[...], preferred_element_type=jnp.float32) + b1_ref[...]
    h = jnp.maximum(h, 0.0)
    o = jnp.dot(h, w2_ref[...], preferred_element_type=jnp.float32) + b2_ref[...]
    d2 = o.shape[1] // 2
    o_ref[0] = o[:, :d2]
    o_ref[1] = o[:, d2:]


def _mlp1(x, w1, b1, w2, b2, a_pad):
    n, d_in = x.shape
    hid = w1.shape[1]
    d_out = w2.shape[1]
    grid = n // MLP_BLK
    return pl.pallas_call(
        _mlp1_body,
        grid=(grid,),
        in_specs=[
            pl.BlockSpec((MLP_BLK, d_in), lambda i: (i, 0)),
            pl.BlockSpec((d_in, hid), lambda i: (0, 0)),
            pl.BlockSpec((1, hid), lambda i: (0, 0)),
            pl.BlockSpec((hid, d_out), lambda i: (0, 0)),
            pl.BlockSpec((1, d_out), lambda i: (0, 0)),
        ],
        out_specs=pl.BlockSpec((2, MLP_BLK, d_out // 2), lambda i: (0, i, 0)),
        out_shape=jax.ShapeDtypeStruct((2, a_pad, d_out // 2), jnp.float32),
    )(x, w1, b1, w2, b2)


def _mlp2_body(p_ref, cf_ref, w1a_ref, w1b_ref, b1_ref, w2_ref, b2_ref, o_ref):
    cembs = jnp.concatenate(
        [jnp.maximum(p_ref[0], 0.0), jnp.maximum(p_ref[1], 0.0)], axis=1)
    h = (jnp.dot(cembs, w1a_ref[...], preferred_element_type=jnp.float32)
         + jnp.dot(cf_ref[...], w1b_ref[...], preferred_element_type=jnp.float32)
         + b1_ref[...])
    h = jnp.maximum(h, 0.0)
    o = jnp.dot(h, w2_ref[...], preferred_element_type=jnp.float32) + b2_ref[...]
    d2 = o.shape[1] // 2
    o_ref[0] = o[:, :d2]
    o_ref[1] = o[:, d2:]


def _mlp2(partials, cf, w1a, w1b, b1, w2, b2, a_pad):
    d2 = partials.shape[2]
    n = cf.shape[0]
    dc = cf.shape[1]
    hid = w1a.shape[1]
    d_out = w2.shape[1]
    grid = n // MLP_BLK
    return pl.pallas_call(
        _mlp2_body,
        grid=(grid,),
        in_specs=[
            pl.BlockSpec((2, MLP_BLK, d2), lambda i: (0, i, 0)),
            pl.BlockSpec((MLP_BLK, dc), lambda i: (i, 0)),
            pl.BlockSpec((2 * d2, hid), lambda i: (0, 0)),
            pl.BlockSpec((dc, hid), lambda i: (0, 0)),
            pl.BlockSpec((1, hid), lambda i: (0, 0)),
            pl.BlockSpec((hid, d_out), lambda i: (0, 0)),
            pl.BlockSpec((1, d_out), lambda i: (0, 0)),
        ],
        out_specs=pl.BlockSpec((2, MLP_BLK, d_out // 2), lambda i: (0, i, 0)),
        out_shape=jax.ShapeDtypeStruct((2, a_pad, d_out // 2), jnp.float32),
    )(partials, cf, w1a, w1b, b1, w2, b2)


def _relu_halves_body(p_ref, o_ref):
    d2 = p_ref.shape[2]
    o_ref[:, :d2] = jnp.maximum(p_ref[0], 0.0)
    o_ref[:, d2:] = jnp.maximum(p_ref[1], 0.0)


def _relu_halves(partials, n):
    d2 = partials.shape[2]
    grid = n // MLP_BLK
    return pl.pallas_call(
        _relu_halves_body,
        grid=(grid,),
        in_specs=[pl.BlockSpec((2, MLP_BLK, d2), lambda i: (0, i, 0))],
        out_specs=pl.BlockSpec((MLP_BLK, 2 * d2), lambda i: (i, 0)),
        out_shape=jax.ShapeDtypeStruct((n, 2 * d2), jnp.float32),
    )(partials)


def _edge_pass(table, g_idx, s_idx, zeros, a_pad):
    d2 = table.shape[2]
    chunks = g_idx.shape[1]
    z_rows = a_pad // NS
    nsub = 4
    qch = chunks // nsub

    def body(table_hbm, g_hbm, s_hbm, z_hbm, out_hbm,
             table_sp, accum, g_v, s_v, rows_v, gsem, ssem):
        c = lax.axis_index("c")
        s = lax.axis_index("s")

        def load_idx(q):
            base = q * qch
            pltpu.async_copy(g_hbm.at[s, pl.ds(base, qch)], g_v, gsem)
            pltpu.async_copy(s_hbm.at[s, pl.ds(base, qch)], s_v, gsem)

        def wait_idx():
            pltpu.make_async_copy(g_hbm.at[s, pl.ds(0, qch)], g_v,
                                  gsem).wait()
            pltpu.make_async_copy(s_hbm.at[s, pl.ds(0, qch)], s_v,
                                  gsem).wait()

        load_idx(0)
        pltpu.sync_copy(z_hbm, accum.at[pl.ds(s * z_rows, z_rows)])
        pltpu.sync_copy(table_hbm.at[c, pl.ds(s * z_rows, z_rows)],
                        table_sp.at[pl.ds(s * z_rows, z_rows)])
        plsc.subcore_barrier()

        def fire_gather(j, b):
            pltpu.async_copy(table_sp.at[g_v.at[j]], rows_v.at[b], gsem)

        def wait_gather(b):
            pltpu.make_async_copy(table_sp.at[g_v.at[0]],
                                  rows_v.at[b], gsem).wait()

        def fire_scatter(j, b):
            pltpu.async_copy(rows_v.at[b], accum.at[s_v.at[j]], ssem,
                             add=True)

        def drain_scatter(b):
            pltpu.make_async_copy(rows_v.at[b], accum.at[s_v.at[0]],
                                  ssem).wait()

        def ring_step(t4, carry):
            for r in range(NB):
                t = t4 * NB + r
                wait_gather(r)
                fire_scatter(t, r)

                @pl.when(t >= 2)
                def _():
                    drain_scatter((r + 2) % NB)

                @pl.when(t + 2 < qch)
                def _():
                    fire_gather(t + 2, (r + 2) % NB)
            return carry

        for q in range(nsub):
            if q:
                load_idx(q)
            wait_idx()
            fire_gather(0, 0)
            fire_gather(1, 1)
            lax.fori_loop(0, qch // NB, ring_step, 0)
            drain_scatter(0)
            drain_scatter(1)

        plsc.subcore_barrier()
        pltpu.sync_copy(accum.at[pl.ds(s * z_rows, z_rows)],
                        out_hbm.at[c, pl.ds(s * z_rows, z_rows)])

    return pl.kernel(
        body,
        out_type=jax.ShapeDtypeStruct((NC, a_pad, d2), jnp.float32),
        mesh=plsc.VectorSubcoreMesh(core_axis_name="c", subcore_axis_name="s"),
        compiler_params=pltpu.CompilerParams(use_tc_tiling_on_sc=False),
        scratch_types=[
            pltpu.VMEM_SHARED((a_pad, d2), jnp.float32),
            pltpu.VMEM_SHARED((a_pad, d2), jnp.float32),
            pltpu.VMEM((chunks // nsub, CHUNK), jnp.int32),
            pltpu.VMEM((chunks // nsub, CHUNK), jnp.int32),
            pltpu.VMEM((NB, CHUNK, d2), jnp.float32),
            pltpu.SemaphoreType.DMA,
            pltpu.SemaphoreType.DMA,
        ],
    )(table, g_idx, s_idx, zeros)


def kernel(literal_feat, clause_feat, edge_index,
           W1_l2c, b1_l2c, W2_l2c, b2_l2c,
           W1_c2l, b1_c2l, W2_c2l, b2_c2l):
    n_lit, d_in = literal_feat.shape
    n_cl = clause_feat.shape[0]
    e = edge_index.shape[1]
    d = W2_l2c.shape[1]
    d2 = d // 2

    grain = NS * CHUNK * 2 * K * 4
    e_pad = -(-e // grain) * grain
    chunks = e_pad // (NS * CHUNK)
    a_pad = -(-(max(n_lit, n_cl) + 1) // (NS * 8)) * (NS * 8)
    dummy = max(n_lit, n_cl)

    src = edge_index[0].astype(jnp.int32)
    dst = edge_index[1].astype(jnp.int32)
    pad = e_pad - e
    dpad = jnp.full((pad,), dummy, jnp.int32)
    srcp = jnp.concatenate([src, dpad]).reshape(NS, chunks, CHUNK)
    dstp = jnp.concatenate([dst, dpad]).reshape(NS, chunks, CHUNK)
    zeros = jnp.zeros((a_pad // NS, d2), jnp.float32)

    wh_l2c = _mlp1(literal_feat, W1_l2c, b1_l2c.reshape(1, -1),
                   W2_l2c, b2_l2c.reshape(1, -1), a_pad)
    p_cl = _edge_pass(wh_l2c, srcp, dstp, zeros, a_pad)
    wh_c2l = _mlp2(p_cl, clause_feat,
                   W1_c2l[:d], W1_c2l[d:],
                   b1_c2l.reshape(1, -1), W2_c2l, b2_c2l.reshape(1, -1), a_pad)
    p_lit = _edge_pass(wh_c2l, dstp, srcp, zeros, a_pad)
    return _relu_halves(p_lit, n_lit)
